# Initial kernel scaffold; baseline (speedup 1.0000x reference)
import jax
import jax.numpy as jnp
from jax.experimental import pallas as pl


def kernel(inputs, edge_types, W_e2h, W_ioux, b_ioux, W_iouh, b_iouh, W_fx, b_fx, W_fh, b_fh):
    raise NotImplementedError("write your pallas kernel here")


# speedup vs baseline: 8.1959x; 8.1959x over previous
import jax
import jax.numpy as jnp
from jax.experimental import pallas as pl
from jax.experimental.pallas import tpu as pltpu

DEPTH = 13
N = 2 ** DEPTH - 1
NI = 2 ** (DEPTH - 1) - 1
NLEAF = N - NI
IN_SIZE = 300
H = 128
T = 37
BLK = 256
LEAF_BLK = 1024

_F32 = jnp.float32


def _tree_body(x_ref, wpair_ref, wflat_ref, wxT_ref, bx_ref, wiouhT_ref,
               wfhT_ref, c_out, h_out, pre_scr, c_scr, h_scr):
    wxT = wxT_ref[...]
    bx = bx_ref[...]

    pre_scr[...] = jnp.dot(x_ref[0:NI, :], wxT,
                           preferred_element_type=_F32) + bx

    for blk in range(NLEAF // LEAF_BLK):
        r0 = NI + blk * LEAF_BLK
        p = jnp.dot(x_ref[r0:r0 + LEAF_BLK, :], wxT,
                    preferred_element_type=_F32) + bx
        i = jax.nn.sigmoid(p[:, 0:H])
        o = jax.nn.sigmoid(p[:, H:2 * H])
        u = jnp.tanh(p[:, 2 * H:3 * H])
        c = i * u
        c_scr[r0:r0 + LEAF_BLK, :] = c
        h_scr[r0:r0 + LEAF_BLK, :] = o * jnp.tanh(c)

    wflat = wflat_ref[...]
    wiouhT = wiouhT_ref[...]
    wfhT = wfhT_ref[...]

    for lev in range(DEPTH - 2, -1, -1):
        m = 1 << lev
        lo = m - 1
        b = min(m, BLK)
        for blk in range(m // b):
            n0 = lo + blk * b
            c0 = 2 * n0 + 1
            ch = h_scr[c0:c0 + 2 * b, :]
            cc = c_scr[c0:c0 + 2 * b, :]
            ch3 = ch.reshape(b, 2, H)
            ch0 = ch3[:, 0, :]
            ch1 = ch3[:, 1, :]
            w0 = wpair_ref[n0:n0 + b, 0:1]
            w1 = wpair_ref[n0:n0 + b, 1:2]
            tcol = jax.lax.broadcasted_iota(jnp.int32, (b, T * H), 1) // H
            a0 = jnp.broadcast_to(ch0[:, None, :], (b, T, H)).reshape(b, T * H)
            a1 = jnp.broadcast_to(ch1[:, None, :], (b, T, H)).reshape(b, T * H)
            A = (a0 * (tcol == w0).astype(_F32)
                 + a1 * (tcol == w1).astype(_F32))
            chs = jnp.dot(A, wflat, preferred_element_type=_F32)
            pre = pre_scr[n0:n0 + b, :]
            iou = pre[:, 0:3 * H] + jnp.dot(chs, wiouhT,
                                            preferred_element_type=_F32)
            i = jax.nn.sigmoid(iou[:, 0:H])
            o = jax.nn.sigmoid(iou[:, H:2 * H])
            u = jnp.tanh(iou[:, 2 * H:3 * H])
            pf = pre[:, 3 * H:4 * H]
            fpre = (jnp.dot(ch, wfhT, preferred_element_type=_F32)
                    + jnp.broadcast_to(pf[:, None, :], (b, 2, H))
                      .reshape(2 * b, H))
            f = jax.nn.sigmoid(fpre)
            fcs = (f * cc).reshape(b, 2, H).sum(axis=1)
            c = i * u + fcs
            h = o * jnp.tanh(c)
            c_scr[n0:n0 + b, :] = c
            h_scr[n0:n0 + b, :] = h

    c_out[...] = c_scr[0:1, :]
    h_out[...] = h_scr[0:1, :]


def _build(interpret=False):
    return pl.pallas_call(
        _tree_body,
        out_shape=[jax.ShapeDtypeStruct((1, H), _F32),
                   jax.ShapeDtypeStruct((1, H), _F32)],
        scratch_shapes=[
            pltpu.VMEM((NI, 4 * H), _F32),
            pltpu.VMEM((N, H), _F32),
            pltpu.VMEM((N, H), _F32),
        ],
        interpret=interpret,
    )


def kernel(inputs, edge_types, W_e2h, W_ioux, b_ioux, W_iouh, b_iouh,
           W_fx, b_fx, W_fh, b_fh):
    wxT = jnp.concatenate([W_ioux, W_fx], axis=0).T
    bx = jnp.concatenate([b_ioux + b_iouh, b_fx + b_fh]).reshape(1, 4 * H)
    wflat = jnp.transpose(W_e2h, (0, 2, 1)).reshape(T * H, H)
    wpair = edge_types[1:].reshape(NI, 2).astype(jnp.int32)
    c, h = _build()(inputs, wpair, wflat, wxT, bx, W_iouh.T, W_fh.T)
    return c, h


# speedup vs baseline: 9.2697x; 1.1310x over previous
import jax
import jax.numpy as jnp
from jax.experimental import pallas as pl
from jax.experimental.pallas import tpu as pltpu

DEPTH = 13
N = 2 ** DEPTH - 1
NI = 2 ** (DEPTH - 1) - 1
NLEAF = N - NI
IN_SIZE = 300
H = 128
T = 37
BLK = 256
LEAF_BLK = 1024

_F32 = jnp.float32
_BF16 = jnp.bfloat16


def _tree_body(x_ref, wpair_ref, wflat_ref, wxT_ref, bx_ref, wiouhT_ref,
               wfhT_ref, c_out, h_out, pre_scr, c_scr, h_scr):
    wxT = wxT_ref[...]
    bx = bx_ref[...]

    pre_scr[...] = jnp.dot(x_ref[0:NI, :], wxT,
                           preferred_element_type=_F32) + bx

    for blk in range(NLEAF // LEAF_BLK):
        r0 = NI + blk * LEAF_BLK
        p = jnp.dot(x_ref[r0:r0 + LEAF_BLK, :], wxT,
                    preferred_element_type=_F32) + bx
        i = jax.nn.sigmoid(p[:, 0:H])
        o = jax.nn.sigmoid(p[:, H:2 * H])
        u = jnp.tanh(p[:, 2 * H:3 * H])
        c = i * u
        c_scr[r0:r0 + LEAF_BLK, :] = c
        h_scr[r0:r0 + LEAF_BLK, :] = o * jnp.tanh(c)

    wflat = wflat_ref[...]
    wiouhT = wiouhT_ref[...]

    for lev in range(DEPTH - 2, -1, -1):
        m = 1 << lev
        lo = m - 1
        b = min(m, BLK)
        for blk in range(m // b):
            n0 = lo + blk * b
            c0 = 2 * n0 + 1
            ch = h_scr[c0:c0 + 2 * b, :].astype(_BF16)
            cc = c_scr[c0:c0 + 2 * b, :]
            ch3 = ch.reshape(b, 2, H)
            ch0 = ch3[:, 0, :]
            ch1 = ch3[:, 1, :]
            w0 = wpair_ref[n0:n0 + b, 0:1]
            w1 = wpair_ref[n0:n0 + b, 1:2]
            tcol = jax.lax.broadcasted_iota(jnp.int32, (b, T * H), 1) // H
            a0 = jnp.broadcast_to(ch0[:, None, :], (b, T, H)).reshape(b, T * H)
            a1 = jnp.broadcast_to(ch1[:, None, :], (b, T, H)).reshape(b, T * H)
            zero = jnp.zeros((), _BF16)
            A = (jnp.where(tcol == w0, a0, zero)
                 + jnp.where(tcol == w1, a1, zero))
            chs = jnp.dot(A, wflat, preferred_element_type=_F32)
            pre = pre_scr[n0:n0 + b, :]
            iou = pre[:, 0:3 * H] + jnp.dot(chs.astype(_BF16), wiouhT,
                                            preferred_element_type=_F32)
            i = jax.nn.sigmoid(iou[:, 0:H])
            o = jax.nn.sigmoid(iou[:, H:2 * H])
            u = jnp.tanh(iou[:, 2 * H:3 * H])
            pf = pre[:, 3 * H:4 * H]
            fpre = (jnp.dot(ch, wfhT_ref[...], preferred_element_type=_F32)
                    + jnp.broadcast_to(pf[:, None, :], (b, 2, H))
                      .reshape(2 * b, H))
            f = jax.nn.sigmoid(fpre)
            fcs = (f * cc).reshape(b, 2, H).sum(axis=1)
            c = i * u + fcs
            h = o * jnp.tanh(c)
            c_scr[n0:n0 + b, :] = c
            h_scr[n0:n0 + b, :] = h

    c_out[...] = c_scr[0:1, :]
    h_out[...] = h_scr[0:1, :]


def _build(interpret=False):
    return pl.pallas_call(
        _tree_body,
        out_shape=[jax.ShapeDtypeStruct((1, H), _F32),
                   jax.ShapeDtypeStruct((1, H), _F32)],
        scratch_shapes=[
            pltpu.VMEM((NI, 4 * H), _F32),
            pltpu.VMEM((N, H), _F32),
            pltpu.VMEM((N, H), _F32),
        ],
        interpret=interpret,
    )


def kernel(inputs, edge_types, W_e2h, W_ioux, b_ioux, W_iouh, b_iouh,
           W_fx, b_fx, W_fh, b_fh):
    wxT = jnp.concatenate([W_ioux, W_fx], axis=0).T.astype(_BF16)
    bx = jnp.concatenate([b_ioux + b_iouh, b_fx + b_fh]).reshape(1, 4 * H)
    wflat = (jnp.transpose(W_e2h, (0, 2, 1)).reshape(T * H, H)
             .astype(_BF16))
    wpair = edge_types[1:].reshape(NI, 2).astype(jnp.int32)
    c, h = _build()(inputs.astype(_BF16), wpair, wflat, wxT, bx,
                    W_iouh.T.astype(_BF16), W_fh.T.astype(_BF16))
    return c, h
